# baseline (speedup 1.0000x reference)
import functools

import jax
import jax.numpy as jnp
from jax import lax
from jax.experimental import pallas as pl
from jax.experimental.pallas import tpu as pltpu
from jax.experimental.pallas import tpu_sc as plsc

_T = 4096
_D = 1024
_NC = 2
_NS = 16
_NW = _NC * _NS
_RPW = _T // _NW
_CH = 32
_NCHUNK = _RPW // _CH


def _make_sc_copy():
    mesh = plsc.VectorSubcoreMesh(core_axis_name="c", subcore_axis_name="s")

    @functools.partial(
        pl.kernel,
        mesh=mesh,
        out_type=jax.ShapeDtypeStruct((_T, _D), jnp.float32),
        scratch_types=[
            pltpu.VMEM((_CH, _D), jnp.float32),
            pltpu.VMEM((_CH, _D), jnp.float32),
            pltpu.SemaphoreType.DMA,
            pltpu.SemaphoreType.DMA,
        ],
    )
    def sc_copy(table_hbm, out_hbm, buf0, buf1, sem0, sem1):
        del buf0, buf1, sem1
        wid = lax.axis_index("s") * _NC + lax.axis_index("c")
        base = wid * _RPW
        pltpu.async_copy(
            table_hbm.at[pl.ds(base, _RPW)], out_hbm.at[pl.ds(base, _RPW)], sem0
        ).wait()

    return sc_copy


_sc_copy = _make_sc_copy()


def kernel(x, pe_table):
    del x
    out = _sc_copy(pe_table)
    return out[None]


# speedup vs baseline: 16.6442x; 16.6442x over previous
import functools

import jax
import jax.numpy as jnp
from jax import lax
from jax.experimental import pallas as pl
from jax.experimental.pallas import tpu as pltpu
from jax.experimental.pallas import tpu_sc as plsc

_T = 4096
_D = 1024
_NC = 2
_NS = 16
_NW = _NC * _NS
_RPW = _T // _NW
_CH = 32
_NCHUNK = _RPW // _CH


_NBUF = 3


def _make_sc_copy():
    mesh = plsc.VectorSubcoreMesh(core_axis_name="c", subcore_axis_name="s")
    n = _RPW // _CH

    @functools.partial(
        pl.kernel,
        mesh=mesh,
        out_type=jax.ShapeDtypeStruct((_T, _D), jnp.float32),
        scratch_types=[
            pltpu.VMEM_SHARED((_NS, _NBUF, _CH, _D), jnp.float32),
            *([pltpu.SemaphoreType.DMA] * (2 * _NBUF)),
        ],
    )
    def sc_copy(table_hbm, out_hbm, stage, *sems):
        in_sems = sems[:_NBUF]
        out_sems = sems[_NBUF:]
        wid = lax.axis_index("s") * _NC + lax.axis_index("c")
        sid = lax.axis_index("s")
        base = wid * _RPW

        def fire_in(k):
            b = k % _NBUF
            return pltpu.async_copy(
                table_hbm.at[pl.ds(base + k * _CH, _CH)], stage.at[sid, b], in_sems[b]
            )

        def fire_out(k):
            b = k % _NBUF
            return pltpu.async_copy(
                stage.at[sid, b], out_hbm.at[pl.ds(base + k * _CH, _CH)], out_sems[b]
            )

        in_cp = [None] * n
        out_cp = [None] * n
        for j in range(min(_NBUF, n)):
            in_cp[j] = fire_in(j)
        for k in range(n):
            if k >= _NBUF:
                out_cp[k - _NBUF].wait()
                in_cp[k] = fire_in(k)
            in_cp[k].wait()
            out_cp[k] = fire_out(k)
        for k in range(max(0, n - _NBUF), n):
            out_cp[k].wait()

    return sc_copy


_sc_copy = _make_sc_copy()


def kernel(x, pe_table):
    del x
    out = _sc_copy(pe_table)
    return out[None]
